# baseline (speedup 1.0000x reference)
import functools

import jax
import jax.numpy as jnp
from jax import lax
from jax.experimental import pallas as pl
from jax.experimental.pallas import tpu as pltpu
from jax.experimental.pallas import tpu_sc as plsc

N = 10000
E = 160000
D_IN = 256
H = 512
L = 5

NSC = 2
NSUB = 16
NCHUNK = 4
CW = H // NCHUNK

NPAD = 10240
ROWS_PER_SUB = NPAD // NSUB

EDGE_BATCH = 120
EPB = 84
IB = 6
EPT = EPB * EDGE_BATCH
E_PAD = EPT * NSUB

_mesh = plsc.VectorSubcoreMesh(core_axis_name="c", subcore_axis_name="s")

CHUNKS_PER_SC = NCHUNK // NSC


@functools.partial(
    pl.kernel,
    out_type=jax.ShapeDtypeStruct((NCHUNK * NPAD, CW), jnp.float32),
    mesh=_mesh,
    scratch_types=[
        pltpu.VMEM((IB * EDGE_BATCH,), jnp.int32),
        pltpu.VMEM((IB * EDGE_BATCH,), jnp.int32),
        pltpu.VMEM((EDGE_BATCH, CW), jnp.float32),
        pltpu.VMEM((EDGE_BATCH, CW), jnp.float32),
        pltpu.VMEM((EDGE_BATCH, CW), jnp.float32),
        pltpu.VMEM_SHARED((NPAD, CW), jnp.float32),
        pltpu.SemaphoreType.DMA,
        pltpu.SemaphoreType.DMA,
        pltpu.SemaphoreType.DMA,
    ],
)
def _segment_sum_sc(hv_hbm, gidx_hbm, dst_hbm, zeros_hbm, out_hbm,
                    gidx_v, dst_v, rows_a, rows_b, rows_c, agg_sh,
                    sem_a, sem_b, sem_c):
    c = lax.axis_index("c")
    s = lax.axis_index("s")
    row0 = s * ROWS_PER_SUB

    @pl.loop(0, CHUNKS_PER_SC)
    def _chunk_pass(r):
        chunk = c * CHUNKS_PER_SC + r
        pltpu.sync_copy(zeros_hbm.at[pl.ds(row0, ROWS_PER_SUB)],
                        agg_sh.at[pl.ds(row0, ROWS_PER_SUB)])
        plsc.subcore_barrier()

        @pl.loop(0, EPB // IB)
        def _idx_block(blk):
            pltpu.sync_copy(
                gidx_hbm.at[pl.ds((chunk * NSUB + s) * EPT
                                  + blk * IB * EDGE_BATCH,
                                  IB * EDGE_BATCH)], gidx_v)
            pltpu.sync_copy(
                dst_hbm.at[pl.ds(s * EPT + blk * IB * EDGE_BATCH,
                                 IB * EDGE_BATCH)], dst_v)

            def _gather(j, buf, sem):
                pltpu.async_copy(
                    hv_hbm.at[gidx_v.at[pl.ds(j * EDGE_BATCH, EDGE_BATCH)]],
                    buf, sem)

            def _wait(buf, sem):
                pltpu.make_async_copy(hv_hbm.at[pl.ds(0, EDGE_BATCH)], buf,
                                      sem).wait()

            def _scatter(j, buf):
                pltpu.sync_copy(
                    buf,
                    agg_sh.at[dst_v.at[pl.ds(j * EDGE_BATCH, EDGE_BATCH)]],
                    add=True)

            _gather(0, rows_a, sem_a)
            _gather(1, rows_b, sem_b)

            @pl.loop(0, IB, step=3)
            def _edge_batch(j):
                _gather(j + 2, rows_c, sem_c)
                _wait(rows_a, sem_a)
                _scatter(j, rows_a)

                @pl.when(j + 3 < IB)
                def _():
                    _gather(j + 3, rows_a, sem_a)

                _wait(rows_b, sem_b)
                _scatter(j + 1, rows_b)

                @pl.when(j + 4 < IB)
                def _():
                    _gather(j + 4, rows_b, sem_b)

                _wait(rows_c, sem_c)
                _scatter(j + 2, rows_c)

        plsc.subcore_barrier()
        pltpu.sync_copy(
            agg_sh.at[pl.ds(row0, ROWS_PER_SUB)],
            out_hbm.at[pl.ds(chunk * NPAD + row0, ROWS_PER_SUB)])


ROW_BLK = 1024


def _bdot(a, b):
    return jnp.dot(a.astype(jnp.bfloat16), b.astype(jnp.bfloat16),
                   preferred_element_type=jnp.float32)


def _enc_body(x_ref, w1_ref, b1_ref, w2_ref, b2_ref, o_ref):
    z = jnp.maximum(_bdot(x_ref[...], w1_ref[...]) + b1_ref[...], 0.0)
    o_ref[...] = (_bdot(z, w2_ref[...])
                  + b2_ref[...]).reshape(NCHUNK * ROW_BLK, CW)


def _encoder_tc(xp, w1, b1, w2, b2):
    grid = (NPAD // ROW_BLK,)
    return pl.pallas_call(
        _enc_body,
        grid=grid,
        in_specs=[
            pl.BlockSpec((ROW_BLK, D_IN), lambda i: (i, 0)),
            pl.BlockSpec((D_IN, H), lambda i: (0, 0)),
            pl.BlockSpec((1, H), lambda i: (0, 0)),
            pl.BlockSpec((H, H), lambda i: (0, 0)),
            pl.BlockSpec((1, H), lambda i: (0, 0)),
        ],
        out_specs=pl.BlockSpec((NCHUNK * ROW_BLK, CW), lambda i: (i, 0)),
        out_shape=jax.ShapeDtypeStruct((NCHUNK * NPAD, CW), jnp.float32),
    )(xp, w1, b1, w2, b2)


def _gin_body(h_ref, agg_ref, w1_ref, b1_ref, w2_ref, b2_ref, eps_ref, o_ref):
    eps1 = 1.0 + eps_ref[0, 0]
    h = h_ref[...].reshape(ROW_BLK, H)
    acc = _bdot(h, w1_ref[...]) * eps1
    for cc in range(NCHUNK):
        acc = acc + _bdot(agg_ref[cc], w1_ref[cc * CW:(cc + 1) * CW, :])
    z = jnp.maximum(acc + b1_ref[...], 0.0)
    out = jnp.maximum(_bdot(z, w2_ref[...]) + b2_ref[...], 0.0)
    o_ref[...] = out.reshape(o_ref.shape)


def _gin_tc(h, agg, w1, b1, w2, b2, eps11, chunked_out=True):
    grid = (NPAD // ROW_BLK,)
    if chunked_out:
        out_spec = pl.BlockSpec((NCHUNK * ROW_BLK, CW), lambda i: (i, 0))
        out_shape = jax.ShapeDtypeStruct((NCHUNK * NPAD, CW), jnp.float32)
    else:
        out_spec = pl.BlockSpec((ROW_BLK, H), lambda i: (i, 0))
        out_shape = jax.ShapeDtypeStruct((NPAD, H), jnp.float32)
    return pl.pallas_call(
        _gin_body,
        grid=grid,
        in_specs=[
            pl.BlockSpec((NCHUNK * ROW_BLK, CW), lambda i: (i, 0)),
            pl.BlockSpec((NCHUNK, ROW_BLK, CW), lambda i: (0, i, 0)),
            pl.BlockSpec((H, H), lambda i: (0, 0)),
            pl.BlockSpec((1, H), lambda i: (0, 0)),
            pl.BlockSpec((H, H), lambda i: (0, 0)),
            pl.BlockSpec((1, H), lambda i: (0, 0)),
            pl.BlockSpec((1, 1), lambda i: (0, 0)),
        ],
        out_specs=out_spec,
        out_shape=out_shape,
    )(h, agg, w1, b1, w2, b2, eps11)


def kernel(x, enc_W1, enc_b1, enc_W2, enc_b2, gin_eps, gin_W1, gin_b1,
           gin_W2, gin_b2, edge_index):
    src = edge_index[0]
    dst = edge_index[1]

    pad_iota = jnp.arange(E_PAD - E, dtype=jnp.int32)
    src_full = jnp.concatenate([src, pad_iota % N])
    dst_full = jnp.concatenate([dst, N + pad_iota % (NPAD - N)])
    gidx = ((src_full * NCHUNK)[None, :]
            + jnp.arange(NCHUNK, dtype=jnp.int32)[:, None])
    gidx = gidx.reshape(-1)
    dstp = dst_full
    zeros = jnp.zeros((NPAD, CW), jnp.float32)

    xp = jnp.pad(x, ((0, NPAD - N), (0, 0)))
    h = _encoder_tc(xp, enc_W1, enc_b1.reshape(1, H), enc_W2,
                    enc_b2.reshape(1, H))

    for l in range(L):
        agg = _segment_sum_sc(h, gidx, dstp, zeros)
        h = _gin_tc(h, agg.reshape(NCHUNK, NPAD, CW), gin_W1[l],
                    gin_b1[l].reshape(1, H), gin_W2[l],
                    gin_b2[l].reshape(1, H), gin_eps[l].reshape(1, 1),
                    chunked_out=(l < L - 1))
    return h[:N]


# speedup vs baseline: 1.1725x; 1.1725x over previous
import functools

import jax
import jax.numpy as jnp
from jax import lax
from jax.experimental import pallas as pl
from jax.experimental.pallas import tpu as pltpu
from jax.experimental.pallas import tpu_sc as plsc

N = 10000
E = 160000
D_IN = 256
H = 512
L = 5

NSC = 2
NSUB = 16
NCHUNK = 4
CW = H // NCHUNK

NPAD = 10240
ROWS_PER_SUB = NPAD // NSUB

EDGE_BATCH = 128
EPB = 80
IB = 20
EPT = EPB * EDGE_BATCH
E_PAD = EPT * NSUB

_mesh = plsc.VectorSubcoreMesh(core_axis_name="c", subcore_axis_name="s")

CHUNKS_PER_SC = NCHUNK // NSC


@functools.partial(
    pl.kernel,
    out_type=jax.ShapeDtypeStruct((NCHUNK * NPAD, CW), jnp.float32),
    mesh=_mesh,
    scratch_types=[
        pltpu.VMEM((IB * EDGE_BATCH,), jnp.int32),
        pltpu.VMEM((IB * EDGE_BATCH,), jnp.int32),
        pltpu.VMEM((EDGE_BATCH, CW), jnp.float32),
        pltpu.VMEM((EDGE_BATCH, CW), jnp.float32),
        pltpu.VMEM_SHARED((NPAD, CW), jnp.float32),
        pltpu.SemaphoreType.DMA,
        pltpu.SemaphoreType.DMA,
    ],
)
def _segment_sum_sc(hv_hbm, gidx_hbm, dst_hbm, zeros_hbm, out_hbm,
                    gidx_v, dst_v, rows_a, rows_b, agg_sh, sem_a, sem_b):
    c = lax.axis_index("c")
    s = lax.axis_index("s")
    row0 = s * ROWS_PER_SUB

    @pl.loop(0, CHUNKS_PER_SC)
    def _chunk_pass(r):
        chunk = c * CHUNKS_PER_SC + r
        pltpu.sync_copy(zeros_hbm.at[pl.ds(row0, ROWS_PER_SUB)],
                        agg_sh.at[pl.ds(row0, ROWS_PER_SUB)])
        plsc.subcore_barrier()

        @pl.loop(0, EPB // IB)
        def _idx_block(blk):
            pltpu.sync_copy(
                gidx_hbm.at[pl.ds((chunk * NSUB + s) * EPT
                                  + blk * IB * EDGE_BATCH,
                                  IB * EDGE_BATCH)], gidx_v)
            pltpu.sync_copy(
                dst_hbm.at[pl.ds(s * EPT + blk * IB * EDGE_BATCH,
                                 IB * EDGE_BATCH)], dst_v)

            def _gather(j, buf, sem):
                pltpu.async_copy(
                    hv_hbm.at[gidx_v.at[pl.ds(j * EDGE_BATCH, EDGE_BATCH)]],
                    buf, sem)

            def _wait(buf, sem):
                pltpu.make_async_copy(hv_hbm.at[pl.ds(0, EDGE_BATCH)], buf,
                                      sem).wait()

            def _scatter(j, buf):
                pltpu.sync_copy(
                    buf,
                    agg_sh.at[dst_v.at[pl.ds(j * EDGE_BATCH, EDGE_BATCH)]],
                    add=True)

            _gather(0, rows_a, sem_a)

            @pl.loop(0, IB, step=2)
            def _edge_batch(j):
                _gather(j + 1, rows_b, sem_b)
                _wait(rows_a, sem_a)
                _scatter(j, rows_a)

                @pl.when(j + 2 < IB)
                def _():
                    _gather(j + 2, rows_a, sem_a)

                _wait(rows_b, sem_b)
                _scatter(j + 1, rows_b)

        plsc.subcore_barrier()
        pltpu.sync_copy(
            agg_sh.at[pl.ds(row0, ROWS_PER_SUB)],
            out_hbm.at[pl.ds(chunk * NPAD + row0, ROWS_PER_SUB)])


ROW_BLK = 2048


def _bdot(a, b):
    return jnp.dot(a.astype(jnp.bfloat16), b.astype(jnp.bfloat16),
                   preferred_element_type=jnp.float32)


def _enc_body(x_ref, w1_ref, b1_ref, w2_ref, b2_ref, o_ref):
    z = jnp.maximum(_bdot(x_ref[...], w1_ref[...]) + b1_ref[...], 0.0)
    o_ref[...] = (_bdot(z, w2_ref[...])
                  + b2_ref[...]).reshape(NCHUNK * ROW_BLK, CW)


def _encoder_tc(xp, w1, b1, w2, b2):
    grid = (NPAD // ROW_BLK,)
    return pl.pallas_call(
        _enc_body,
        grid=grid,
        in_specs=[
            pl.BlockSpec((ROW_BLK, D_IN), lambda i: (i, 0)),
            pl.BlockSpec((D_IN, H), lambda i: (0, 0)),
            pl.BlockSpec((1, H), lambda i: (0, 0)),
            pl.BlockSpec((H, H), lambda i: (0, 0)),
            pl.BlockSpec((1, H), lambda i: (0, 0)),
        ],
        out_specs=pl.BlockSpec((NCHUNK * ROW_BLK, CW), lambda i: (i, 0)),
        out_shape=jax.ShapeDtypeStruct((NCHUNK * NPAD, CW), jnp.float32),
    )(xp, w1, b1, w2, b2)


def _gin_body(h_ref, agg_ref, w1_ref, b1_ref, w2_ref, b2_ref, eps_ref, o_ref):
    eps1 = 1.0 + eps_ref[0, 0]
    h = h_ref[...].reshape(ROW_BLK, H)
    acc = _bdot(h, w1_ref[...]) * eps1
    for cc in range(NCHUNK):
        acc = acc + _bdot(agg_ref[cc], w1_ref[cc * CW:(cc + 1) * CW, :])
    z = jnp.maximum(acc + b1_ref[...], 0.0)
    out = jnp.maximum(_bdot(z, w2_ref[...]) + b2_ref[...], 0.0)
    o_ref[...] = out.reshape(o_ref.shape)


def _gin_tc(h, agg, w1, b1, w2, b2, eps11, chunked_out=True):
    grid = (NPAD // ROW_BLK,)
    if chunked_out:
        out_spec = pl.BlockSpec((NCHUNK * ROW_BLK, CW), lambda i: (i, 0))
        out_shape = jax.ShapeDtypeStruct((NCHUNK * NPAD, CW), jnp.float32)
    else:
        out_spec = pl.BlockSpec((ROW_BLK, H), lambda i: (i, 0))
        out_shape = jax.ShapeDtypeStruct((NPAD, H), jnp.float32)
    return pl.pallas_call(
        _gin_body,
        grid=grid,
        in_specs=[
            pl.BlockSpec((NCHUNK * ROW_BLK, CW), lambda i: (i, 0)),
            pl.BlockSpec((NCHUNK, ROW_BLK, CW), lambda i: (0, i, 0)),
            pl.BlockSpec((H, H), lambda i: (0, 0)),
            pl.BlockSpec((1, H), lambda i: (0, 0)),
            pl.BlockSpec((H, H), lambda i: (0, 0)),
            pl.BlockSpec((1, H), lambda i: (0, 0)),
            pl.BlockSpec((1, 1), lambda i: (0, 0)),
        ],
        out_specs=out_spec,
        out_shape=out_shape,
    )(h, agg, w1, b1, w2, b2, eps11)


def kernel(x, enc_W1, enc_b1, enc_W2, enc_b2, gin_eps, gin_W1, gin_b1,
           gin_W2, gin_b2, edge_index):
    src = edge_index[0]
    dst = edge_index[1]

    pad_iota = jnp.arange(E_PAD - E, dtype=jnp.int32)
    src_full = jnp.concatenate([src, pad_iota % N])
    dst_full = jnp.concatenate([dst, N + pad_iota % (NPAD - N)])
    gidx = ((src_full * NCHUNK)[None, :]
            + jnp.arange(NCHUNK, dtype=jnp.int32)[:, None])
    gidx = gidx.reshape(-1)
    dstp = dst_full
    zeros = jnp.zeros((NPAD, CW), jnp.float32)

    xp = jnp.pad(x, ((0, NPAD - N), (0, 0)))
    h = _encoder_tc(xp, enc_W1, enc_b1.reshape(1, H), enc_W2,
                    enc_b2.reshape(1, H))

    for l in range(L):
        agg = _segment_sum_sc(h, gidx, dstp, zeros)
        h = _gin_tc(h, agg.reshape(NCHUNK, NPAD, CW), gin_W1[l],
                    gin_b1[l].reshape(1, H), gin_W2[l],
                    gin_b2[l].reshape(1, H), gin_eps[l].reshape(1, 1),
                    chunked_out=(l < L - 1))
    return h[:N]


# speedup vs baseline: 1.2246x; 1.0445x over previous
import functools

import jax
import jax.numpy as jnp
from jax import lax
from jax.experimental import pallas as pl
from jax.experimental.pallas import tpu as pltpu
from jax.experimental.pallas import tpu_sc as plsc

N = 10000
E = 160000
D_IN = 256
H = 512
L = 5

NSC = 2
NSUB = 16
NCHUNK = 4
CW = H // NCHUNK

NPAD = 10240
ROWS_PER_SUB = NPAD // NSUB

EDGE_BATCH = 128
EPB = 80
IB = 40
EPT = EPB * EDGE_BATCH
E_PAD = EPT * NSUB

_mesh = plsc.VectorSubcoreMesh(core_axis_name="c", subcore_axis_name="s")

CHUNKS_PER_SC = NCHUNK // NSC


@functools.partial(
    pl.kernel,
    out_type=jax.ShapeDtypeStruct((NCHUNK * NPAD, CW), jnp.float32),
    mesh=_mesh,
    scratch_types=[
        pltpu.VMEM((IB * EDGE_BATCH,), jnp.int32),
        pltpu.VMEM((IB * EDGE_BATCH,), jnp.int32),
        pltpu.VMEM((EDGE_BATCH, CW), jnp.float32),
        pltpu.VMEM((EDGE_BATCH, CW), jnp.float32),
        pltpu.VMEM_SHARED((NPAD, CW), jnp.float32),
        pltpu.SemaphoreType.DMA,
        pltpu.SemaphoreType.DMA,
    ],
)
def _segment_sum_sc(hv_hbm, gidx_hbm, dst_hbm, zeros_hbm, out_hbm,
                    gidx_v, dst_v, rows_a, rows_b, agg_sh, sem_a, sem_b):
    c = lax.axis_index("c")
    s = lax.axis_index("s")
    row0 = s * ROWS_PER_SUB

    @pl.loop(0, CHUNKS_PER_SC)
    def _chunk_pass(r):
        chunk = c * CHUNKS_PER_SC + r
        pltpu.sync_copy(zeros_hbm.at[pl.ds(row0, ROWS_PER_SUB)],
                        agg_sh.at[pl.ds(row0, ROWS_PER_SUB)])
        plsc.subcore_barrier()

        @pl.loop(0, EPB // IB)
        def _idx_block(blk):
            pltpu.sync_copy(
                gidx_hbm.at[pl.ds((chunk * NSUB + s) * EPT
                                  + blk * IB * EDGE_BATCH,
                                  IB * EDGE_BATCH)], gidx_v)
            pltpu.sync_copy(
                dst_hbm.at[pl.ds(s * EPT + blk * IB * EDGE_BATCH,
                                 IB * EDGE_BATCH)], dst_v)

            def _gather(j, buf, sem):
                pltpu.async_copy(
                    hv_hbm.at[gidx_v.at[pl.ds(j * EDGE_BATCH, EDGE_BATCH)]],
                    buf, sem)

            def _wait(buf, sem):
                pltpu.make_async_copy(hv_hbm.at[pl.ds(0, EDGE_BATCH)], buf,
                                      sem).wait()

            def _scatter(j, buf):
                pltpu.sync_copy(
                    buf,
                    agg_sh.at[dst_v.at[pl.ds(j * EDGE_BATCH, EDGE_BATCH)]],
                    add=True)

            _gather(0, rows_a, sem_a)

            @pl.loop(0, IB, step=2)
            def _edge_batch(j):
                _gather(j + 1, rows_b, sem_b)
                _wait(rows_a, sem_a)
                _scatter(j, rows_a)

                @pl.when(j + 2 < IB)
                def _():
                    _gather(j + 2, rows_a, sem_a)

                _wait(rows_b, sem_b)
                _scatter(j + 1, rows_b)

        plsc.subcore_barrier()
        pltpu.sync_copy(
            agg_sh.at[pl.ds(row0, ROWS_PER_SUB)],
            out_hbm.at[pl.ds(chunk * NPAD + row0, ROWS_PER_SUB)])


ROW_BLK = 2048


def _bdot(a, b):
    return jnp.dot(a.astype(jnp.bfloat16), b.astype(jnp.bfloat16),
                   preferred_element_type=jnp.float32)


def _enc_body(x_ref, w1_ref, b1_ref, w2_ref, b2_ref, o_ref):
    z = jnp.maximum(_bdot(x_ref[...], w1_ref[...]) + b1_ref[...], 0.0)
    o_ref[...] = (_bdot(z, w2_ref[...])
                  + b2_ref[...]).reshape(NCHUNK * ROW_BLK, CW)


def _encoder_tc(xp, w1, b1, w2, b2):
    grid = (NPAD // ROW_BLK,)
    return pl.pallas_call(
        _enc_body,
        grid=grid,
        in_specs=[
            pl.BlockSpec((ROW_BLK, D_IN), lambda i: (i, 0)),
            pl.BlockSpec((D_IN, H), lambda i: (0, 0)),
            pl.BlockSpec((1, H), lambda i: (0, 0)),
            pl.BlockSpec((H, H), lambda i: (0, 0)),
            pl.BlockSpec((1, H), lambda i: (0, 0)),
        ],
        out_specs=pl.BlockSpec((NCHUNK * ROW_BLK, CW), lambda i: (i, 0)),
        out_shape=jax.ShapeDtypeStruct((NCHUNK * NPAD, CW), jnp.float32),
    )(xp, w1, b1, w2, b2)


def _gin_body(h_ref, agg_ref, w1_ref, b1_ref, w2_ref, b2_ref, eps_ref, o_ref):
    eps1 = 1.0 + eps_ref[0, 0]
    h = h_ref[...].reshape(ROW_BLK, H)
    acc = _bdot(h, w1_ref[...]) * eps1
    for cc in range(NCHUNK):
        acc = acc + _bdot(agg_ref[cc], w1_ref[cc * CW:(cc + 1) * CW, :])
    z = jnp.maximum(acc + b1_ref[...], 0.0)
    out = jnp.maximum(_bdot(z, w2_ref[...]) + b2_ref[...], 0.0)
    o_ref[...] = out.reshape(o_ref.shape)


def _gin_tc(h, agg, w1, b1, w2, b2, eps11, chunked_out=True):
    grid = (NPAD // ROW_BLK,)
    if chunked_out:
        out_spec = pl.BlockSpec((NCHUNK * ROW_BLK, CW), lambda i: (i, 0))
        out_shape = jax.ShapeDtypeStruct((NCHUNK * NPAD, CW), jnp.float32)
    else:
        out_spec = pl.BlockSpec((ROW_BLK, H), lambda i: (i, 0))
        out_shape = jax.ShapeDtypeStruct((NPAD, H), jnp.float32)
    return pl.pallas_call(
        _gin_body,
        grid=grid,
        in_specs=[
            pl.BlockSpec((NCHUNK * ROW_BLK, CW), lambda i: (i, 0)),
            pl.BlockSpec((NCHUNK, ROW_BLK, CW), lambda i: (0, i, 0)),
            pl.BlockSpec((H, H), lambda i: (0, 0)),
            pl.BlockSpec((1, H), lambda i: (0, 0)),
            pl.BlockSpec((H, H), lambda i: (0, 0)),
            pl.BlockSpec((1, H), lambda i: (0, 0)),
            pl.BlockSpec((1, 1), lambda i: (0, 0)),
        ],
        out_specs=out_spec,
        out_shape=out_shape,
    )(h, agg, w1, b1, w2, b2, eps11)


def kernel(x, enc_W1, enc_b1, enc_W2, enc_b2, gin_eps, gin_W1, gin_b1,
           gin_W2, gin_b2, edge_index):
    src = edge_index[0]
    dst = edge_index[1]

    pad_iota = jnp.arange(E_PAD - E, dtype=jnp.int32)
    src_full = jnp.concatenate([src, pad_iota % N])
    dst_full = jnp.concatenate([dst, N + pad_iota % (NPAD - N)])
    gidx = ((src_full * NCHUNK)[None, :]
            + jnp.arange(NCHUNK, dtype=jnp.int32)[:, None])
    gidx = gidx.reshape(-1)
    dstp = dst_full
    zeros = jnp.zeros((NPAD, CW), jnp.float32)

    xp = jnp.pad(x, ((0, NPAD - N), (0, 0)))
    h = _encoder_tc(xp, enc_W1, enc_b1.reshape(1, H), enc_W2,
                    enc_b2.reshape(1, H))

    for l in range(L):
        agg = _segment_sum_sc(h, gidx, dstp, zeros)
        h = _gin_tc(h, agg.reshape(NCHUNK, NPAD, CW), gin_W1[l],
                    gin_b1[l].reshape(1, H), gin_W2[l],
                    gin_b2[l].reshape(1, H), gin_eps[l].reshape(1, 1),
                    chunked_out=(l < L - 1))
    return h[:N]
